# Initial kernel scaffold; baseline (speedup 1.0000x reference)
import jax
import jax.numpy as jnp
from jax.experimental import pallas as pl


def kernel(visual_input, query_input, memory_keys, W_enc, b_enc, W0, b0, ln_gamma, ln_beta, W_cls, b_cls):
    raise NotImplementedError("write your pallas kernel here")


# speedup vs baseline: 16.0895x; 16.0895x over previous
import functools

import jax
import jax.numpy as jnp
from jax import lax
from jax.experimental import pallas as pl
from jax.experimental.pallas import tpu as pltpu
from jax.experimental.pallas import tpu_sc as plsc

B = 1024
K = 100000
D = 128
U = 128
NC = 1000
TOPK = 128

CW = 128
KP = 102400
C = KP // CW
KT = 4096
NKT = KP // KT

NEG = -1e30

SC_CORES = 2
SC_SUBCORES = 16
NW = SC_CORES * SC_SUBCORES
RPW = B // NW
GB = 64
CAND = 2048


def _f32_to_key(bits_i32):
  return bits_i32 ^ ((bits_i32 >> 31) & jnp.int32(0x7FFFFFFF))


def _k1_body(q_ref, wenc_ref, benc_ref, w0_ref, b0_ref, mk_ref,
             sim_ref, cm_ref, qemb_ref, qt_ref, qn_s):
  i = pl.program_id(0)

  @pl.when(i == 0)
  def _prep():
    q = q_ref[...]
    qe = jnp.maximum(
        lax.dot_general(q, wenc_ref[...], (((1,), (1,)), ((), ())))
        + benc_ref[...], 0.0)
    qemb_ref[...] = qe
    nrm = jnp.sqrt(jnp.sum(qe * qe, axis=1, keepdims=True))
    qn_s[...] = qe / jnp.maximum(nrm, 1e-8)
    qt_ref[...] = jnp.maximum(
        lax.dot_general(qe, w0_ref[...], (((1,), (1,)), ((), ())))
        + b0_ref[...], 0.0)

  mk = mk_ref[...]
  nrm = jnp.sqrt(jnp.sum(mk * mk, axis=1, keepdims=True))
  kn = mk / jnp.maximum(nrm, 1e-8)
  s = lax.dot_general(qn_s[...], kn, (((1,), (1,)), ((), ())))

  @pl.when(i == NKT - 1)
  def _mask_pad():
    col = i * KT + lax.broadcasted_iota(jnp.int32, (B, KT), 1)
    sim_ref[...] = jnp.where(col < K, s, NEG)

  @pl.when(i < NKT - 1)
  def _store_plain():
    sim_ref[...] = s

  sv = sim_ref[...]
  cm_ref[...] = jnp.max(sv.reshape(B, KT // CW, CW),
                        axis=2)[None]


def _k1b_body(cm_ref, t0_ref):
  cm = cm_ref[...]
  key = _f32_to_key(lax.bitcast_convert_type(cm, jnp.int32))
  cnt0 = jnp.sum((key >= 0).astype(jnp.int32), axis=1, keepdims=True)
  t = jnp.where(cnt0 >= TOPK, jnp.int32(0), jnp.int32(-2147483648))
  for bit in range(30, -1, -1):
    cand = t + jnp.int32(1 << bit)
    cnt = jnp.sum((key >= cand).astype(jnp.int32), axis=1, keepdims=True)
    t = jnp.where(cnt >= TOPK, cand, t)
  fbits = _f32_to_key(t)
  t0 = lax.bitcast_convert_type(fbits, jnp.float32)
  t0_ref[...] = jnp.broadcast_to(t0, (B, 16))


def _run_k1(query_input, W_enc, b_enc, W0, b0, memory_keys):
  return pl.pallas_call(
      _k1_body,
      grid=(NKT,),
      in_specs=[
          pl.BlockSpec((B, D), lambda i: (0, 0)),
          pl.BlockSpec((D, D), lambda i: (0, 0)),
          pl.BlockSpec((1, D), lambda i: (0, 0)),
          pl.BlockSpec((U, D), lambda i: (0, 0)),
          pl.BlockSpec((1, U), lambda i: (0, 0)),
          pl.BlockSpec((KT, D), lambda i: (i, 0)),
      ],
      out_specs=[
          pl.BlockSpec((B, KT), lambda i: (0, i)),
          pl.BlockSpec((1, B, KT // CW), lambda i: (i, 0, 0)),
          pl.BlockSpec((B, D), lambda i: (0, 0)),
          pl.BlockSpec((B, U), lambda i: (0, 0)),
      ],
      out_shape=[
          jax.ShapeDtypeStruct((B, KP), jnp.float32),
          jax.ShapeDtypeStruct((NKT, B, KT // CW), jnp.float32),
          jax.ShapeDtypeStruct((B, D), jnp.float32),
          jax.ShapeDtypeStruct((B, U), jnp.float32),
      ],
      scratch_shapes=[
          pltpu.VMEM((B, D), jnp.float32),
      ],
      compiler_params=pltpu.CompilerParams(
          dimension_semantics=("arbitrary",)),
  )(query_input, W_enc, b_enc, W0, b0, memory_keys)


def _run_k1b(cm):
  return pl.pallas_call(
      _k1b_body,
      in_specs=[pl.BlockSpec((B, C), lambda: (0, 0))],
      out_specs=pl.BlockSpec((B, 16), lambda: (0, 0)),
      out_shape=jax.ShapeDtypeStruct((B, 16), jnp.float32),
  )(cm)


_INT_MIN = -2147483648


def _sc_body(sim2, cm, t0, qt, keys, att_out,
             cm_v, t0_v, qt_v, selid_v, chunkbuf, candk_v, candp_v,
             keysbuf, idx_v, att_v, sem):
  cix = lax.axis_index("c")
  six = lax.axis_index("s")
  wid = six * SC_CORES + cix
  base = wid * RPW

  pltpu.sync_copy(cm.at[pl.ds(base, RPW)], cm_v)
  pltpu.sync_copy(t0.at[pl.ds(base, RPW)], t0_v)
  pltpu.sync_copy(qt.at[pl.ds(base, RPW)], qt_v)

  iota16 = lax.iota(jnp.int32, 16)
  zero16i = jnp.zeros((16,), jnp.int32)

  def scatter_append(ref, base, x, m):
    pos = base + plsc.cumsum(m.astype(jnp.int32)) - 1
    pos = jnp.maximum(pos, 0)
    plsc.store_scatter(ref, [pos], x, mask=m)

  def row_body(i, _):
    r = base + i
    t0vec = t0_v[i, pl.ds(0, 16)]

    def sel_body(j, nsel):
      v = cm_v[i, pl.ds(j * 16, 16)]
      m = v >= t0vec
      cnt = jnp.sum(m.astype(jnp.int32))

      @pl.when(cnt > 0)
      def _():
        ids = (r * C + j * 16) + iota16
        scatter_append(selid_v, nsel, ids, m)

      return nsel + cnt

    nsel = lax.fori_loop(0, C // 16, sel_body, jnp.int32(0), unroll=False)

    safeid = jnp.full((16,), r * C + (C - 1), jnp.int32)
    for pj in range(4):
      plsc.store_scatter(selid_v, [nsel + pj * 16 + iota16], safeid)

    def refine(ncand, thr_f):
      del thr_f
      plsc.store_scatter(candk_v, [ncand + iota16],
                         zero16i + jnp.int32(_INT_MIN))
      nv = (ncand + 15) // 16

      def cnt0_body(j, c):
        kv = candk_v[pl.ds(j * 16, 16)]
        return c + jnp.sum((kv >= 0).astype(jnp.int32))

      cnt0 = lax.fori_loop(0, nv, cnt0_body, jnp.int32(0), unroll=False)
      t = jnp.where(cnt0 >= TOPK, jnp.int32(0), jnp.int32(_INT_MIN))

      def bit_step(bi, t):
        bit = 31 - bi
        candt = t + (jnp.int32(1) << bit)

        def cnt_body(j, c):
          kv = candk_v[pl.ds(j * 16, 16)]
          return c + jnp.sum((kv >= candt).astype(jnp.int32))

        cnt = lax.fori_loop(0, nv, cnt_body, jnp.int32(0), unroll=False)
        return jnp.where(cnt >= TOPK, candt, t)

      t = lax.fori_loop(1, 32, bit_step, t, unroll=False)

      def cgt_body(j, c):
        kv = candk_v[pl.ds(j * 16, 16)]
        return c + jnp.sum((kv >= t + 1).astype(jnp.int32))

      c_gt = lax.fori_loop(0, nv, cgt_body, jnp.int32(0), unroll=False)
      fill = TOPK - c_gt

      def extract_body(j, carry):
        woff, eqcnt = carry
        kv = candk_v[pl.ds(j * 16, 16)]
        pv = candp_v[pl.ds(j * 16, 16)]
        gt = kv >= (t + 1)
        eq = kv == t
        pos = plsc.cumsum(eq.astype(jnp.int32)) + eqcnt
        keep = gt | (eq & (pos <= fill))
        cnt = jnp.sum(keep.astype(jnp.int32))

        @pl.when(cnt > 0)
        def _():
          scatter_append(candk_v, woff, kv, keep)
          scatter_append(candp_v, woff, pv, keep)

        return woff + cnt, eqcnt + jnp.sum(eq.astype(jnp.int32))

      _, _ = lax.fori_loop(0, nv, extract_body,
                           (jnp.int32(0), jnp.int32(0)), unroll=False)
      newthr = lax.bitcast_convert_type(
          _f32_to_key(jnp.zeros((16,), jnp.int32) + t), jnp.float32)
      return jnp.int32(TOPK), newthr

    def batch_cond(carry):
      off, ncand, thr_f = carry
      return off < nsel

    def batch_body(carry):
      off, ncand, thr_f = carry
      aoff = pl.multiple_of(off, GB)
      pltpu.async_copy(sim2.at[selid_v.at[pl.ds(aoff, GB)]], chunkbuf,
                       sem).wait()

      def vreg_body(j, inner):
        ncand, thr_f = inner
        g = j // 8
        kk = j % 8
        v = chunkbuf[g, pl.ds(kk * 16, 16)]
        m = v >= thr_f
        cnt = jnp.sum(m.astype(jnp.int32))

        def do_store(args):
          ncand, thr_f = args
          ncand, thr_f = lax.cond(ncand + 16 > CAND, refine,
                                  lambda n, tf: (n, tf), ncand, thr_f)
          kv = _f32_to_key(lax.bitcast_convert_type(
              chunkbuf[g, pl.ds(kk * 16, 16)], jnp.int32))
          m2 = chunkbuf[g, pl.ds(kk * 16, 16)] >= thr_f
          pv = (off + g) * CW + kk * 16 + iota16
          scatter_append(candk_v, ncand, kv, m2)
          scatter_append(candp_v, ncand, pv, m2)
          return ncand + jnp.sum(m2.astype(jnp.int32)), thr_f

        return lax.cond(cnt > 0, do_store, lambda a: a, (ncand, thr_f))

      ncand, thr_f = lax.fori_loop(0, GB * 8, vreg_body, (ncand, thr_f),
                                   unroll=False)
      return off + GB, ncand, thr_f

    thr0 = t0vec
    off0 = jnp.int32(0)
    _, ncand, thr_f = lax.while_loop(
        batch_cond, batch_body, (off0, jnp.int32(0), thr0))

    ncand, thr_f = refine(ncand, thr_f)

    for dd in range(8):
      p = candp_v[pl.ds(dd * 16, 16)]
      slot = p >> 7
      lane = p & jnp.int32(127)
      cid = plsc.load_gather(selid_v, [slot]) - r * C
      idx_v[pl.ds(dd * 16, 16)] = cid * CW + lane

    pltpu.async_copy(keys.at[idx_v], keysbuf, sem).wait()

    qtv = [qt_v[i, pl.ds(dd * 16, 16)] for dd in range(8)]

    svs = []
    for kg in range(8):
      def score_body(j, sv, kg=kg):
        kk2 = kg * 16 + j
        acc = qtv[0] * keysbuf[kk2, pl.ds(0, 16)]
        for dd in range(1, 8):
          acc = acc + qtv[dd] * keysbuf[kk2, pl.ds(dd * 16, 16)]
        s = jnp.sum(acc)
        return jnp.where(iota16 == j, s, sv)

      svs.append(lax.fori_loop(0, 16, score_body,
                               jnp.zeros((16,), jnp.float32), unroll=False))

    mxv = svs[0]
    for kg in range(1, 8):
      mxv = jnp.maximum(mxv, svs[kg])
    mxv = jnp.zeros((16,), jnp.float32) + jnp.max(mxv)

    evs = [jnp.exp(sv - mxv) for sv in svs]
    ssum = evs[0]
    for kg in range(1, 8):
      ssum = ssum + evs[kg]
    rzv = 1.0 / (jnp.zeros((16,), jnp.float32) + jnp.sum(ssum))
    wvs = [ev * rzv for ev in evs]

    accs = tuple(jnp.zeros((16,), jnp.float32) for _ in range(8))
    for kg in range(8):
      def att_body(j, accs, kg=kg):
        kk2 = kg * 16 + j
        ws = jnp.take(wvs[kg], jnp.full((16,), j, jnp.int32))
        return tuple(accs[dd] + ws * keysbuf[kk2, pl.ds(dd * 16, 16)]
                     for dd in range(8))

      accs = lax.fori_loop(0, 16, att_body, accs, unroll=False)
    for dd in range(8):
      att_v[i, pl.ds(dd * 16, 16)] = accs[dd]
    return 0

  lax.fori_loop(0, RPW, row_body, 0, unroll=False)
  pltpu.sync_copy(att_v, att_out.at[pl.ds(base, RPW)])


def _run_sc(sim2, cm, t0, qt, memory_keys):
  mesh = plsc.VectorSubcoreMesh(
      core_axis_name="c", subcore_axis_name="s",
      num_cores=SC_CORES, num_subcores=SC_SUBCORES)
  fn = pl.kernel(
      _sc_body,
      out_type=jax.ShapeDtypeStruct((B, D), jnp.float32),
      mesh=mesh,
      compiler_params=pltpu.CompilerParams(needs_layout_passes=False),
      scratch_types=[
          pltpu.VMEM((RPW, C), jnp.float32),
          pltpu.VMEM((RPW, 16), jnp.float32),
          pltpu.VMEM((RPW, U), jnp.float32),
          pltpu.VMEM((C + GB,), jnp.int32),
          pltpu.VMEM((GB, CW), jnp.float32),
          pltpu.VMEM((CAND + 16,), jnp.int32),
          pltpu.VMEM((CAND + 16,), jnp.int32),
          pltpu.VMEM((TOPK, D), jnp.float32),
          pltpu.VMEM((TOPK,), jnp.int32),
          pltpu.VMEM((RPW, D), jnp.float32),
          pltpu.SemaphoreType.DMA,
      ],
  )
  return fn(sim2, cm, t0, qt, memory_keys)


def _k3_body(qemb_ref, att_ref, g_ref, b_ref, wcls_ref, bcls_ref, out_ref):
  qe = qemb_ref[...]
  x = att_ref[...] + qe
  mu = jnp.mean(x, axis=1, keepdims=True)
  xc = x - mu
  var = jnp.mean(xc * xc, axis=1, keepdims=True)
  ln = xc * lax.rsqrt(var + 1e-5) * g_ref[...] + b_ref[...]
  merged = jnp.concatenate([qe, ln], axis=1)
  out_ref[...] = lax.dot_general(
      merged, wcls_ref[...], (((1,), (1,)), ((), ()))) + bcls_ref[...]


def _run_k3(qemb, att, ln_gamma, ln_beta, W_cls, b_cls):
  return pl.pallas_call(
      _k3_body,
      in_specs=[
          pl.BlockSpec((B, D), lambda: (0, 0)),
          pl.BlockSpec((B, D), lambda: (0, 0)),
          pl.BlockSpec((1, D), lambda: (0, 0)),
          pl.BlockSpec((1, D), lambda: (0, 0)),
          pl.BlockSpec((NC, 2 * D), lambda: (0, 0)),
          pl.BlockSpec((1, NC), lambda: (0, 0)),
      ],
      out_specs=pl.BlockSpec((B, NC), lambda: (0, 0)),
      out_shape=jax.ShapeDtypeStruct((B, NC), jnp.float32),
  )(qemb, att, ln_gamma.reshape(1, D), ln_beta.reshape(1, D),
    W_cls, b_cls.reshape(1, NC))


def kernel(visual_input, query_input, memory_keys, W_enc, b_enc, W0, b0,
           ln_gamma, ln_beta, W_cls, b_cls):
  del visual_input
  sim, cm3, qemb, qt = _run_k1(
      query_input, W_enc, b_enc.reshape(1, D), W0, b0.reshape(1, U),
      memory_keys)
  cm = cm3.transpose(1, 0, 2).reshape(B, C)
  t0 = _run_k1b(cm)
  sim2 = sim.reshape(B * C, CW)
  att = _run_sc(sim2, cm, t0, qt, memory_keys)
  return _run_k3(qemb, att, ln_gamma, ln_beta, W_cls, b_cls)


# speedup vs baseline: 17.2972x; 1.0751x over previous
import functools

import jax
import jax.numpy as jnp
from jax import lax
from jax.experimental import pallas as pl
from jax.experimental.pallas import tpu as pltpu
from jax.experimental.pallas import tpu_sc as plsc

B = 1024
K = 100000
D = 128
U = 128
NC = 1000
TOPK = 128

CW = 128
CSH = 7
KP = 102400
C = KP // CW
KT = 4096
NKT = KP // KT
VPC = CW // 16

NEG = -1e30

SC_CORES = 2
SC_SUBCORES = 16
NW = SC_CORES * SC_SUBCORES
RPW = B // NW
GB = 64
CAND = 2048


def _f32_to_key(bits_i32):
  return bits_i32 ^ ((bits_i32 >> 31) & jnp.int32(0x7FFFFFFF))


def _k1_body(q_ref, wenc_ref, benc_ref, w0_ref, b0_ref, mk_ref,
             sim_ref, cm_ref, qemb_ref, qt_ref, qn_s):
  i = pl.program_id(0)

  @pl.when(i == 0)
  def _prep():
    q = q_ref[...]
    qe = jnp.maximum(
        lax.dot_general(q, wenc_ref[...], (((1,), (1,)), ((), ())))
        + benc_ref[...], 0.0)
    qemb_ref[...] = qe
    nrm = jnp.sqrt(jnp.sum(qe * qe, axis=1, keepdims=True))
    qn_s[...] = qe / jnp.maximum(nrm, 1e-8)
    qt_ref[...] = jnp.maximum(
        lax.dot_general(qe, w0_ref[...], (((1,), (1,)), ((), ())))
        + b0_ref[...], 0.0)

  mk = mk_ref[...]
  nrm = jnp.sqrt(jnp.sum(mk * mk, axis=1, keepdims=True))
  kn = mk / jnp.maximum(nrm, 1e-8)
  s = lax.dot_general(qn_s[...], kn, (((1,), (1,)), ((), ())))

  @pl.when(i == NKT - 1)
  def _mask_pad():
    col = i * KT + lax.broadcasted_iota(jnp.int32, (B, KT), 1)
    sim_ref[...] = jnp.where(col < K, s, NEG)

  @pl.when(i < NKT - 1)
  def _store_plain():
    sim_ref[...] = s

  sv = sim_ref[...]
  cm_ref[...] = jnp.max(sv.reshape(B, KT // CW, CW),
                        axis=2)[None]


def _k1b_body(cm_ref, t0_ref):
  cm = cm_ref[...]
  key = _f32_to_key(lax.bitcast_convert_type(cm, jnp.int32))
  cnt0 = jnp.sum((key >= 0).astype(jnp.int32), axis=1, keepdims=True)
  t = jnp.where(cnt0 >= TOPK, jnp.int32(0), jnp.int32(-2147483648))
  for bit in range(30, -1, -1):
    cand = t + jnp.int32(1 << bit)
    cnt = jnp.sum((key >= cand).astype(jnp.int32), axis=1, keepdims=True)
    t = jnp.where(cnt >= TOPK, cand, t)
  fbits = _f32_to_key(t)
  t0 = lax.bitcast_convert_type(fbits, jnp.float32)
  t0_ref[...] = jnp.broadcast_to(t0, (B, 16))


def _run_k1(query_input, W_enc, b_enc, W0, b0, memory_keys):
  return pl.pallas_call(
      _k1_body,
      grid=(NKT,),
      in_specs=[
          pl.BlockSpec((B, D), lambda i: (0, 0)),
          pl.BlockSpec((D, D), lambda i: (0, 0)),
          pl.BlockSpec((1, D), lambda i: (0, 0)),
          pl.BlockSpec((U, D), lambda i: (0, 0)),
          pl.BlockSpec((1, U), lambda i: (0, 0)),
          pl.BlockSpec((KT, D), lambda i: (i, 0)),
      ],
      out_specs=[
          pl.BlockSpec((B, KT), lambda i: (0, i)),
          pl.BlockSpec((1, B, KT // CW), lambda i: (i, 0, 0)),
          pl.BlockSpec((B, D), lambda i: (0, 0)),
          pl.BlockSpec((B, U), lambda i: (0, 0)),
      ],
      out_shape=[
          jax.ShapeDtypeStruct((B, KP), jnp.float32),
          jax.ShapeDtypeStruct((NKT, B, KT // CW), jnp.float32),
          jax.ShapeDtypeStruct((B, D), jnp.float32),
          jax.ShapeDtypeStruct((B, U), jnp.float32),
      ],
      scratch_shapes=[
          pltpu.VMEM((B, D), jnp.float32),
      ],
      compiler_params=pltpu.CompilerParams(
          dimension_semantics=("arbitrary",)),
  )(query_input, W_enc, b_enc, W0, b0, memory_keys)


def _run_k1b(cm):
  return pl.pallas_call(
      _k1b_body,
      in_specs=[pl.BlockSpec((B, C), lambda: (0, 0))],
      out_specs=pl.BlockSpec((B, 16), lambda: (0, 0)),
      out_shape=jax.ShapeDtypeStruct((B, 16), jnp.float32),
  )(cm)


_INT_MIN = -2147483648


def _sc_body(sim2, cm, t0, qt, keys, att_out,
             cm_v, t0_v, qt_v, selid_v, chunkbuf, candk_v, candp_v,
             keysbuf, idx_v, att_v, sem):
  cix = lax.axis_index("c")
  six = lax.axis_index("s")
  wid = six * SC_CORES + cix
  base = wid * RPW

  pltpu.sync_copy(cm.at[pl.ds(base, RPW)], cm_v)
  pltpu.sync_copy(t0.at[pl.ds(base, RPW)], t0_v)
  pltpu.sync_copy(qt.at[pl.ds(base, RPW)], qt_v)

  iota16 = lax.iota(jnp.int32, 16)
  zero16i = jnp.zeros((16,), jnp.int32)

  def scatter_append(ref, base, x, m):
    pos = base + plsc.cumsum(m.astype(jnp.int32)) - 1
    pos = jnp.maximum(pos, 0)
    plsc.store_scatter(ref, [pos], x, mask=m)

  def row_body(i, _):
    r = base + i
    t0vec = t0_v[i, pl.ds(0, 16)]

    def sel_body(j, nsel):
      v = cm_v[i, pl.ds(j * 16, 16)]
      m = v >= t0vec
      cnt = jnp.sum(m.astype(jnp.int32))

      @pl.when(cnt > 0)
      def _():
        ids = (r * C + j * 16) + iota16
        scatter_append(selid_v, nsel, ids, m)

      return nsel + cnt

    nsel = lax.fori_loop(0, C // 16, sel_body, jnp.int32(0), unroll=False)

    safeid = jnp.full((16,), r * C + (C - 1), jnp.int32)
    for pj in range(4):
      plsc.store_scatter(selid_v, [nsel + pj * 16 + iota16], safeid)

    def refine(ncand, thr_f):
      del thr_f
      plsc.store_scatter(candk_v, [ncand + iota16],
                         zero16i + jnp.int32(_INT_MIN))
      nv = (ncand + 15) // 16

      def mm_body(j, c):
        mn, mx = c
        kv = candk_v[pl.ds(j * 16, 16)]
        kvm = jnp.where(kv == jnp.int32(_INT_MIN),
                        jnp.int32(2147483647), kv)
        return jnp.minimum(mn, kvm), jnp.maximum(mx, kv)

      mnv, mxv = lax.fori_loop(
          0, nv, mm_body,
          (jnp.full((16,), 2147483647, jnp.int32),
           jnp.full((16,), _INT_MIN, jnp.int32)), unroll=False)
      kmin = jnp.min(mnv)
      kmax = jnp.max(mxv)

      def bit_step(bi, t):
        step = jnp.int32(1) << (30 - bi)

        def probe(t):
          candt = t + step

          def cnt_body(j, cacc):
            kv = candk_v[pl.ds(j * 16, 16)]
            return cacc + (kv >= candt).astype(jnp.int32)

          cacc = lax.fori_loop(0, nv, cnt_body, zero16i, unroll=False)
          cnt = jnp.sum(cacc)
          return jnp.where(cnt >= TOPK, candt, t)

        return lax.cond(step <= kmax - t, probe, lambda t: t, t)

      t = lax.fori_loop(0, 31, bit_step, kmin, unroll=False)

      def cgt_body(j, c):
        kv = candk_v[pl.ds(j * 16, 16)]
        return c + (kv >= t + 1).astype(jnp.int32)

      c_gt = jnp.sum(lax.fori_loop(0, nv, cgt_body, zero16i, unroll=False))
      fill = TOPK - c_gt

      def extract_body(j, carry):
        woff, eqcnt = carry
        kv = candk_v[pl.ds(j * 16, 16)]
        pv = candp_v[pl.ds(j * 16, 16)]
        gt = kv >= (t + 1)
        eq = kv == t
        pos = plsc.cumsum(eq.astype(jnp.int32)) + eqcnt
        keep = gt | (eq & (pos <= fill))
        cnt = jnp.sum(keep.astype(jnp.int32))

        @pl.when(cnt > 0)
        def _():
          scatter_append(candk_v, woff, kv, keep)
          scatter_append(candp_v, woff, pv, keep)

        return woff + cnt, eqcnt + jnp.sum(eq.astype(jnp.int32))

      _, _ = lax.fori_loop(0, nv, extract_body,
                           (jnp.int32(0), jnp.int32(0)), unroll=False)
      newthr = lax.bitcast_convert_type(
          _f32_to_key(jnp.zeros((16,), jnp.int32) + t), jnp.float32)
      return jnp.int32(TOPK), newthr

    def batch_cond(carry):
      off, ncand, thr_f = carry
      return off < nsel

    def batch_body(carry):
      off, ncand, thr_f = carry
      aoff = pl.multiple_of(off, GB)
      pltpu.async_copy(sim2.at[selid_v.at[pl.ds(aoff, GB)]], chunkbuf,
                       sem).wait()

      def chunk_body(g, inner):
        carry2 = inner
        for kk in range(VPC):
          v = chunkbuf[g, pl.ds(kk * 16, 16)]
          hit = jnp.any(v >= carry2[1])

          def do_store(args, v=v, kk=kk):
            ncand, thr_f = args
            ncand, thr_f = lax.cond(ncand + 16 > CAND, refine,
                                    lambda n, tf: (n, tf), ncand, thr_f)
            m2 = v >= thr_f
            kv = _f32_to_key(lax.bitcast_convert_type(v, jnp.int32))
            pv = (off + g) * CW + kk * 16 + iota16
            scatter_append(candk_v, ncand, kv, m2)
            scatter_append(candp_v, ncand, pv, m2)
            return ncand + jnp.sum(m2.astype(jnp.int32)), thr_f

          carry2 = lax.cond(hit, do_store, lambda a: a, carry2)
        return carry2

      ncand, thr_f = lax.fori_loop(0, GB, chunk_body, (ncand, thr_f),
                                   unroll=False)
      return off + GB, ncand, thr_f

    thr0 = t0vec
    off0 = jnp.int32(0)
    _, ncand, thr_f = lax.while_loop(
        batch_cond, batch_body, (off0, jnp.int32(0), thr0))

    ncand, thr_f = refine(ncand, thr_f)

    for dd in range(8):
      p = candp_v[pl.ds(dd * 16, 16)]
      slot = p >> CSH
      lane = p & jnp.int32(CW - 1)
      cid = plsc.load_gather(selid_v, [slot]) - r * C
      idx_v[pl.ds(dd * 16, 16)] = cid * CW + lane

    pltpu.async_copy(keys.at[idx_v], keysbuf, sem).wait()

    qtv = [qt_v[i, pl.ds(dd * 16, 16)] for dd in range(8)]

    svs = []
    for kg in range(8):
      def score_body(j, sv, kg=kg):
        kk2 = kg * 16 + j
        acc = qtv[0] * keysbuf[kk2, pl.ds(0, 16)]
        for dd in range(1, 8):
          acc = acc + qtv[dd] * keysbuf[kk2, pl.ds(dd * 16, 16)]
        s = jnp.sum(acc)
        return jnp.where(iota16 == j, s, sv)

      svs.append(lax.fori_loop(0, 16, score_body,
                               jnp.zeros((16,), jnp.float32), unroll=4))

    mxv = svs[0]
    for kg in range(1, 8):
      mxv = jnp.maximum(mxv, svs[kg])
    mxv = jnp.zeros((16,), jnp.float32) + jnp.max(mxv)

    evs = [jnp.exp(sv - mxv) for sv in svs]
    ssum = evs[0]
    for kg in range(1, 8):
      ssum = ssum + evs[kg]
    rzv = 1.0 / (jnp.zeros((16,), jnp.float32) + jnp.sum(ssum))
    wvs = [ev * rzv for ev in evs]

    accs = tuple(jnp.zeros((16,), jnp.float32) for _ in range(8))
    for kg in range(8):
      def att_body(j, accs, kg=kg):
        kk2 = kg * 16 + j
        ws = jnp.take(wvs[kg], jnp.full((16,), j, jnp.int32))
        return tuple(accs[dd] + ws * keysbuf[kk2, pl.ds(dd * 16, 16)]
                     for dd in range(8))

      accs = lax.fori_loop(0, 16, att_body, accs, unroll=4)
    for dd in range(8):
      att_v[i, pl.ds(dd * 16, 16)] = accs[dd]
    return 0

  lax.fori_loop(0, RPW, row_body, 0, unroll=False)
  pltpu.sync_copy(att_v, att_out.at[pl.ds(base, RPW)])


def _run_sc(sim2, cm, t0, qt, memory_keys):
  mesh = plsc.VectorSubcoreMesh(
      core_axis_name="c", subcore_axis_name="s",
      num_cores=SC_CORES, num_subcores=SC_SUBCORES)
  fn = pl.kernel(
      _sc_body,
      out_type=jax.ShapeDtypeStruct((B, D), jnp.float32),
      mesh=mesh,
      compiler_params=pltpu.CompilerParams(needs_layout_passes=False),
      scratch_types=[
          pltpu.VMEM((RPW, C), jnp.float32),
          pltpu.VMEM((RPW, 16), jnp.float32),
          pltpu.VMEM((RPW, U), jnp.float32),
          pltpu.VMEM((C + GB,), jnp.int32),
          pltpu.VMEM((GB, CW), jnp.float32),
          pltpu.VMEM((CAND + 16,), jnp.int32),
          pltpu.VMEM((CAND + 16,), jnp.int32),
          pltpu.VMEM((TOPK, D), jnp.float32),
          pltpu.VMEM((TOPK,), jnp.int32),
          pltpu.VMEM((RPW, D), jnp.float32),
          pltpu.SemaphoreType.DMA,
      ],
  )
  return fn(sim2, cm, t0, qt, memory_keys)


def _k3_body(qemb_ref, att_ref, g_ref, b_ref, wcls_ref, bcls_ref, out_ref):
  qe = qemb_ref[...]
  x = att_ref[...] + qe
  mu = jnp.mean(x, axis=1, keepdims=True)
  xc = x - mu
  var = jnp.mean(xc * xc, axis=1, keepdims=True)
  ln = xc * lax.rsqrt(var + 1e-5) * g_ref[...] + b_ref[...]
  merged = jnp.concatenate([qe, ln], axis=1)
  out_ref[...] = lax.dot_general(
      merged, wcls_ref[...], (((1,), (1,)), ((), ()))) + bcls_ref[...]


def _run_k3(qemb, att, ln_gamma, ln_beta, W_cls, b_cls):
  return pl.pallas_call(
      _k3_body,
      in_specs=[
          pl.BlockSpec((B, D), lambda: (0, 0)),
          pl.BlockSpec((B, D), lambda: (0, 0)),
          pl.BlockSpec((1, D), lambda: (0, 0)),
          pl.BlockSpec((1, D), lambda: (0, 0)),
          pl.BlockSpec((NC, 2 * D), lambda: (0, 0)),
          pl.BlockSpec((1, NC), lambda: (0, 0)),
      ],
      out_specs=pl.BlockSpec((B, NC), lambda: (0, 0)),
      out_shape=jax.ShapeDtypeStruct((B, NC), jnp.float32),
  )(qemb, att, ln_gamma.reshape(1, D), ln_beta.reshape(1, D),
    W_cls, b_cls.reshape(1, NC))


def kernel(visual_input, query_input, memory_keys, W_enc, b_enc, W0, b0,
           ln_gamma, ln_beta, W_cls, b_cls):
  del visual_input
  sim, cm3, qemb, qt = _run_k1(
      query_input, W_enc, b_enc.reshape(1, D), W0, b0.reshape(1, U),
      memory_keys)
  cm = cm3.transpose(1, 0, 2).reshape(B, C)
  t0 = _run_k1b(cm)
  sim2 = sim.reshape(B * C, CW)
  att = _run_sc(sim2, cm, t0, qt, memory_keys)
  return _run_k3(qemb, att, ln_gamma, ln_beta, W_cls, b_cls)


# speedup vs baseline: 20.0767x; 1.1607x over previous
import functools

import jax
import jax.numpy as jnp
from jax import lax
from jax.experimental import pallas as pl
from jax.experimental.pallas import tpu as pltpu
from jax.experimental.pallas import tpu_sc as plsc

B = 1024
K = 100000
D = 128
U = 128
NC = 1000
TOPK = 128

CW = 128
CSH = 7
KP = 102400
C = KP // CW
KT = 4096
NKT = KP // KT
VPC = CW // 16

NEG = -1e30

SC_CORES = 2
SC_SUBCORES = 16
NW = SC_CORES * SC_SUBCORES
RPW = B // NW
GB = 64
CAND = 2048


def _f32_to_key(bits_i32):
  return bits_i32 ^ ((bits_i32 >> 31) & jnp.int32(0x7FFFFFFF))


def _k1_body(q_ref, wenc_ref, benc_ref, w0_ref, b0_ref, mk_ref,
             sim_ref, cm_ref, qemb_ref, qt_ref, qn_s):
  i = pl.program_id(0)

  @pl.when(i == 0)
  def _prep():
    q = q_ref[...]
    qe = jnp.maximum(
        lax.dot_general(q, wenc_ref[...], (((1,), (1,)), ((), ())))
        + benc_ref[...], 0.0)
    qemb_ref[...] = qe
    nrm = jnp.sqrt(jnp.sum(qe * qe, axis=1, keepdims=True))
    qn_s[...] = qe / jnp.maximum(nrm, 1e-8)
    qt_ref[...] = jnp.maximum(
        lax.dot_general(qe, w0_ref[...], (((1,), (1,)), ((), ())))
        + b0_ref[...], 0.0)

  mk = mk_ref[...]
  nrm = jnp.sqrt(jnp.sum(mk * mk, axis=1, keepdims=True))
  kn = mk / jnp.maximum(nrm, 1e-8)
  s = lax.dot_general(qn_s[...], kn, (((1,), (1,)), ((), ())))

  b = qn_s.shape[0]

  @pl.when(i == NKT - 1)
  def _mask_pad():
    col = i * KT + lax.broadcasted_iota(jnp.int32, (b, KT), 1)
    sim_ref[...] = jnp.where(col < K, s, NEG)

  @pl.when(i < NKT - 1)
  def _store_plain():
    sim_ref[...] = s

  sv = sim_ref[...]
  cm_ref[...] = jnp.max(sv.reshape(b, KT // CW, CW),
                        axis=2)[None]


def _k1b_body(cm_ref, t0_ref):
  cm = cm_ref[...]
  b = cm.shape[0]
  key = _f32_to_key(lax.bitcast_convert_type(cm, jnp.int32))
  cnt0 = jnp.sum((key >= 0).astype(jnp.int32), axis=1, keepdims=True)
  t = jnp.where(cnt0 >= TOPK, jnp.int32(0), jnp.int32(-2147483648))
  for bit in range(30, -1, -1):
    cand = t + jnp.int32(1 << bit)
    cnt = jnp.sum((key >= cand).astype(jnp.int32), axis=1, keepdims=True)
    t = jnp.where(cnt >= TOPK, cand, t)
  fbits = _f32_to_key(t)
  t0 = lax.bitcast_convert_type(fbits, jnp.float32)
  t0_ref[...] = jnp.broadcast_to(t0, (b, 16))


def _run_k1(query_input, W_enc, b_enc, W0, b0, memory_keys, b):
  return pl.pallas_call(
      _k1_body,
      grid=(NKT,),
      in_specs=[
          pl.BlockSpec((b, D), lambda i: (0, 0)),
          pl.BlockSpec((D, D), lambda i: (0, 0)),
          pl.BlockSpec((1, D), lambda i: (0, 0)),
          pl.BlockSpec((U, D), lambda i: (0, 0)),
          pl.BlockSpec((1, U), lambda i: (0, 0)),
          pl.BlockSpec((KT, D), lambda i: (i, 0)),
      ],
      out_specs=[
          pl.BlockSpec((b, KT), lambda i: (0, i)),
          pl.BlockSpec((1, b, KT // CW), lambda i: (i, 0, 0)),
          pl.BlockSpec((b, D), lambda i: (0, 0)),
          pl.BlockSpec((b, U), lambda i: (0, 0)),
      ],
      out_shape=[
          jax.ShapeDtypeStruct((b, KP), jnp.float32),
          jax.ShapeDtypeStruct((NKT, b, KT // CW), jnp.float32),
          jax.ShapeDtypeStruct((b, D), jnp.float32),
          jax.ShapeDtypeStruct((b, U), jnp.float32),
      ],
      scratch_shapes=[
          pltpu.VMEM((b, D), jnp.float32),
      ],
      compiler_params=pltpu.CompilerParams(
          dimension_semantics=("arbitrary",)),
  )(query_input, W_enc, b_enc, W0, b0, memory_keys)


def _run_k1b(cm, b):
  return pl.pallas_call(
      _k1b_body,
      in_specs=[pl.BlockSpec((b, C), lambda: (0, 0))],
      out_specs=pl.BlockSpec((b, 16), lambda: (0, 0)),
      out_shape=jax.ShapeDtypeStruct((b, 16), jnp.float32),
  )(cm)


_INT_MIN = -2147483648


def _sc_body(sim2, cm, t0, qt, keys, att_out,
             cm_v, t0_v, qt_v, selid_v, chunkbuf, candk_v, candp_v,
             keysbuf, idx_v, att_v, sem):
  rpw = att_v.shape[0]
  cix = lax.axis_index("c")
  six = lax.axis_index("s")
  wid = six * SC_CORES + cix
  base = wid * rpw

  pltpu.sync_copy(cm.at[pl.ds(base, rpw)], cm_v)
  pltpu.sync_copy(t0.at[pl.ds(base, rpw)], t0_v)
  pltpu.sync_copy(qt.at[pl.ds(base, rpw)], qt_v)

  iota16 = lax.iota(jnp.int32, 16)
  zero16i = jnp.zeros((16,), jnp.int32)

  def scatter_append(ref, base, x, m):
    pos = base + plsc.cumsum(m.astype(jnp.int32)) - 1
    pos = jnp.maximum(pos, 0)
    plsc.store_scatter(ref, [pos], x, mask=m)

  def row_body(i, _):
    r = base + i
    t0vec = t0_v[i, pl.ds(0, 16)]

    def sel_body(j, nsel):
      v = cm_v[i, pl.ds(j * 16, 16)]
      m = v >= t0vec
      cnt = jnp.sum(m.astype(jnp.int32))

      @pl.when(cnt > 0)
      def _():
        ids = (r * C + j * 16) + iota16
        scatter_append(selid_v, nsel, ids, m)

      return nsel + cnt

    nsel = lax.fori_loop(0, C // 16, sel_body, jnp.int32(0), unroll=False)

    safeid = jnp.full((16,), r * C + (C - 1), jnp.int32)
    for pj in range(4):
      plsc.store_scatter(selid_v, [nsel + pj * 16 + iota16], safeid)

    def refine(ncand, thr_f):
      del thr_f
      plsc.store_scatter(candk_v, [ncand + iota16],
                         zero16i + jnp.int32(_INT_MIN))
      nv = (ncand + 15) // 16

      def mm_body(j, c):
        mn, mx = c
        kv = candk_v[pl.ds(j * 16, 16)]
        kvm = jnp.where(kv == jnp.int32(_INT_MIN),
                        jnp.int32(2147483647), kv)
        return jnp.minimum(mn, kvm), jnp.maximum(mx, kv)

      mnv, mxv = lax.fori_loop(
          0, nv, mm_body,
          (jnp.full((16,), 2147483647, jnp.int32),
           jnp.full((16,), _INT_MIN, jnp.int32)), unroll=False)
      kmin = jnp.min(mnv)
      kmax = jnp.max(mxv)

      def bit_step(bi, t):
        step = jnp.int32(1) << (30 - bi)

        def probe(t):
          candt = t + step

          def cnt_body(j, cacc):
            kv = candk_v[pl.ds(j * 16, 16)]
            return cacc + (kv >= candt).astype(jnp.int32)

          cacc = lax.fori_loop(0, nv, cnt_body, zero16i, unroll=False)
          cnt = jnp.sum(cacc)
          return jnp.where(cnt >= TOPK, candt, t)

        return lax.cond(step <= kmax - t, probe, lambda t: t, t)

      t = lax.fori_loop(0, 31, bit_step, kmin, unroll=False)

      def cgt_body(j, c):
        kv = candk_v[pl.ds(j * 16, 16)]
        return c + (kv >= t + 1).astype(jnp.int32)

      c_gt = jnp.sum(lax.fori_loop(0, nv, cgt_body, zero16i, unroll=False))
      fill = TOPK - c_gt

      def extract_body(j, carry):
        woff, eqcnt = carry
        kv = candk_v[pl.ds(j * 16, 16)]
        pv = candp_v[pl.ds(j * 16, 16)]
        gt = kv >= (t + 1)
        eq = kv == t
        pos = plsc.cumsum(eq.astype(jnp.int32)) + eqcnt
        keep = gt | (eq & (pos <= fill))
        cnt = jnp.sum(keep.astype(jnp.int32))

        @pl.when(cnt > 0)
        def _():
          scatter_append(candk_v, woff, kv, keep)
          scatter_append(candp_v, woff, pv, keep)

        return woff + cnt, eqcnt + jnp.sum(eq.astype(jnp.int32))

      _, _ = lax.fori_loop(0, nv, extract_body,
                           (jnp.int32(0), jnp.int32(0)), unroll=False)
      newthr = lax.bitcast_convert_type(
          _f32_to_key(jnp.zeros((16,), jnp.int32) + t), jnp.float32)
      return jnp.int32(TOPK), newthr

    def batch_cond(carry):
      off, ncand, thr_f = carry
      return off < nsel

    def batch_body(carry):
      off, ncand, thr_f = carry
      aoff = pl.multiple_of(off, GB)
      pltpu.async_copy(sim2.at[selid_v.at[pl.ds(aoff, GB)]], chunkbuf,
                       sem).wait()

      def chunk_body(g, inner):
        carry2 = inner
        for kk in range(VPC):
          v = chunkbuf[g, pl.ds(kk * 16, 16)]
          hit = jnp.any(v >= carry2[1])

          def do_store(args, v=v, kk=kk):
            ncand, thr_f = args
            ncand, thr_f = lax.cond(ncand + 16 > CAND, refine,
                                    lambda n, tf: (n, tf), ncand, thr_f)
            m2 = v >= thr_f
            kv = _f32_to_key(lax.bitcast_convert_type(v, jnp.int32))
            pv = (off + g) * CW + kk * 16 + iota16
            scatter_append(candk_v, ncand, kv, m2)
            scatter_append(candp_v, ncand, pv, m2)
            return ncand + jnp.sum(m2.astype(jnp.int32)), thr_f

          carry2 = lax.cond(hit, do_store, lambda a: a, carry2)
        return carry2

      ncand, thr_f = lax.fori_loop(0, GB, chunk_body, (ncand, thr_f),
                                   unroll=False)
      return off + GB, ncand, thr_f

    thr0 = t0vec
    off0 = jnp.int32(0)
    _, ncand, thr_f = lax.while_loop(
        batch_cond, batch_body, (off0, jnp.int32(0), thr0))

    ncand, thr_f = refine(ncand, thr_f)

    for dd in range(8):
      p = candp_v[pl.ds(dd * 16, 16)]
      slot = p >> CSH
      lane = p & jnp.int32(CW - 1)
      cid = plsc.load_gather(selid_v, [slot]) - r * C
      idx_v[pl.ds(dd * 16, 16)] = cid * CW + lane

    pltpu.async_copy(keys.at[idx_v], keysbuf, sem).wait()

    qtv = [qt_v[i, pl.ds(dd * 16, 16)] for dd in range(8)]

    svs = []
    for kg in range(8):
      def score_body(j, sv, kg=kg):
        kk2 = kg * 16 + j
        acc = qtv[0] * keysbuf[kk2, pl.ds(0, 16)]
        for dd in range(1, 8):
          acc = acc + qtv[dd] * keysbuf[kk2, pl.ds(dd * 16, 16)]
        s = jnp.sum(acc)
        return jnp.where(iota16 == j, s, sv)

      svs.append(lax.fori_loop(0, 16, score_body,
                               jnp.zeros((16,), jnp.float32), unroll=4))

    mxv = svs[0]
    for kg in range(1, 8):
      mxv = jnp.maximum(mxv, svs[kg])
    mxv = jnp.zeros((16,), jnp.float32) + jnp.max(mxv)

    evs = [jnp.exp(sv - mxv) for sv in svs]
    ssum = evs[0]
    for kg in range(1, 8):
      ssum = ssum + evs[kg]
    rzv = 1.0 / (jnp.zeros((16,), jnp.float32) + jnp.sum(ssum))
    wvs = [ev * rzv for ev in evs]

    accs = tuple(jnp.zeros((16,), jnp.float32) for _ in range(8))
    for kg in range(8):
      def att_body(j, accs, kg=kg):
        kk2 = kg * 16 + j
        ws = jnp.take(wvs[kg], jnp.full((16,), j, jnp.int32))
        return tuple(accs[dd] + ws * keysbuf[kk2, pl.ds(dd * 16, 16)]
                     for dd in range(8))

      accs = lax.fori_loop(0, 16, att_body, accs, unroll=4)
    for dd in range(8):
      att_v[i, pl.ds(dd * 16, 16)] = accs[dd]
    return 0

  lax.fori_loop(0, rpw, row_body, 0, unroll=False)
  pltpu.sync_copy(att_v, att_out.at[pl.ds(base, rpw)])


def _run_sc(sim2, cm, t0, qt, memory_keys, b):
  rpw = b // NW
  mesh = plsc.VectorSubcoreMesh(
      core_axis_name="c", subcore_axis_name="s",
      num_cores=SC_CORES, num_subcores=SC_SUBCORES)
  fn = pl.kernel(
      _sc_body,
      out_type=jax.ShapeDtypeStruct((b, D), jnp.float32),
      mesh=mesh,
      compiler_params=pltpu.CompilerParams(needs_layout_passes=False),
      scratch_types=[
          pltpu.VMEM((rpw, C), jnp.float32),
          pltpu.VMEM((rpw, 16), jnp.float32),
          pltpu.VMEM((rpw, U), jnp.float32),
          pltpu.VMEM((C + GB,), jnp.int32),
          pltpu.VMEM((GB, CW), jnp.float32),
          pltpu.VMEM((CAND + 16,), jnp.int32),
          pltpu.VMEM((CAND + 16,), jnp.int32),
          pltpu.VMEM((TOPK, D), jnp.float32),
          pltpu.VMEM((TOPK,), jnp.int32),
          pltpu.VMEM((rpw, D), jnp.float32),
          pltpu.SemaphoreType.DMA,
      ],
  )
  return fn(sim2, cm, t0, qt, memory_keys)


def _k3_body(qemb_ref, att_ref, g_ref, b_ref, wcls_ref, bcls_ref, out_ref):
  qe = qemb_ref[...]
  x = att_ref[...] + qe
  mu = jnp.mean(x, axis=1, keepdims=True)
  xc = x - mu
  var = jnp.mean(xc * xc, axis=1, keepdims=True)
  ln = xc * lax.rsqrt(var + 1e-5) * g_ref[...] + b_ref[...]
  merged = jnp.concatenate([qe, ln], axis=1)
  out_ref[...] = lax.dot_general(
      merged, wcls_ref[...], (((1,), (1,)), ((), ()))) + bcls_ref[...]


def _run_k3(qemb, att, ln_gamma, ln_beta, W_cls, b_cls, b):
  return pl.pallas_call(
      _k3_body,
      in_specs=[
          pl.BlockSpec((b, D), lambda: (0, 0)),
          pl.BlockSpec((b, D), lambda: (0, 0)),
          pl.BlockSpec((1, D), lambda: (0, 0)),
          pl.BlockSpec((1, D), lambda: (0, 0)),
          pl.BlockSpec((NC, 2 * D), lambda: (0, 0)),
          pl.BlockSpec((1, NC), lambda: (0, 0)),
      ],
      out_specs=pl.BlockSpec((b, NC), lambda: (0, 0)),
      out_shape=jax.ShapeDtypeStruct((b, NC), jnp.float32),
  )(qemb, att, ln_gamma.reshape(1, D), ln_beta.reshape(1, D),
    W_cls, b_cls.reshape(1, NC))


NSPLIT = 2


def kernel(visual_input, query_input, memory_keys, W_enc, b_enc, W0, b0,
           ln_gamma, ln_beta, W_cls, b_cls):
  del visual_input
  bh = B // NSPLIT
  outs = []
  for h in range(NSPLIT):
    q = lax.slice_in_dim(query_input, h * bh, (h + 1) * bh, axis=0)
    sim, cm3, qemb, qt = _run_k1(
        q, W_enc, b_enc.reshape(1, D), W0, b0.reshape(1, U),
        memory_keys, bh)
    cm = cm3.transpose(1, 0, 2).reshape(bh, C)
    t0 = _run_k1b(cm, bh)
    sim2 = sim.reshape(bh * C, CW)
    att = _run_sc(sim2, cm, t0, qt, memory_keys, bh)
    outs.append(_run_k3(qemb, att, ln_gamma, ln_beta, W_cls, b_cls, bh))
  return jnp.concatenate(outs, axis=0)


# speedup vs baseline: 21.2984x; 1.0609x over previous
import functools

import jax
import jax.numpy as jnp
from jax import lax
from jax.experimental import pallas as pl
from jax.experimental.pallas import tpu as pltpu
from jax.experimental.pallas import tpu_sc as plsc

B = 1024
K = 100000
D = 128
U = 128
NC = 1000
TOPK = 128

CW = 128
CSH = 7
KP = 102400
C = KP // CW
KT = 4096
NKT = KP // KT
VPC = CW // 16

NEG = -1e30

SC_CORES = 2
SC_SUBCORES = 16
NW = SC_CORES * SC_SUBCORES
RPW = B // NW
GB = 64
CAND = 2048


def _f32_to_key(bits_i32):
  return bits_i32 ^ ((bits_i32 >> 31) & jnp.int32(0x7FFFFFFF))


def _k1_body(q_ref, wenc_ref, benc_ref, w0_ref, b0_ref, mk_ref,
             sim_ref, cm_ref, qemb_ref, qt_ref, qn_s):
  i = pl.program_id(0)

  @pl.when(i == 0)
  def _prep():
    q = q_ref[...]
    qe = jnp.maximum(
        lax.dot_general(q, wenc_ref[...], (((1,), (1,)), ((), ())))
        + benc_ref[...], 0.0)
    qemb_ref[...] = qe
    nrm = jnp.sqrt(jnp.sum(qe * qe, axis=1, keepdims=True))
    qn_s[...] = qe / jnp.maximum(nrm, 1e-8)
    qt_ref[...] = jnp.maximum(
        lax.dot_general(qe, w0_ref[...], (((1,), (1,)), ((), ())))
        + b0_ref[...], 0.0)

  mk = mk_ref[...]
  nrm = jnp.sqrt(jnp.sum(mk * mk, axis=1, keepdims=True))
  kn = mk / jnp.maximum(nrm, 1e-8)
  s = lax.dot_general(qn_s[...], kn, (((1,), (1,)), ((), ())))

  b = qn_s.shape[0]

  @pl.when(i == NKT - 1)
  def _mask_pad():
    col = i * KT + lax.broadcasted_iota(jnp.int32, (b, KT), 1)
    sim_ref[...] = jnp.where(col < K, s, NEG)

  @pl.when(i < NKT - 1)
  def _store_plain():
    sim_ref[...] = s

  sv = sim_ref[...]
  cm_ref[...] = jnp.max(sv.reshape(b, KT // CW, CW),
                        axis=2)[None]


def _k1b_body(cm_ref, t0_ref):
  cm = cm_ref[...]
  b = cm.shape[0]
  key = _f32_to_key(lax.bitcast_convert_type(cm, jnp.int32))
  cnt0 = jnp.sum((key >= 0).astype(jnp.int32), axis=1, keepdims=True)
  t = jnp.where(cnt0 >= TOPK, jnp.int32(0), jnp.int32(-2147483648))
  for bit in range(30, -1, -1):
    cand = t + jnp.int32(1 << bit)
    cnt = jnp.sum((key >= cand).astype(jnp.int32), axis=1, keepdims=True)
    t = jnp.where(cnt >= TOPK, cand, t)
  fbits = _f32_to_key(t)
  t0 = lax.bitcast_convert_type(fbits, jnp.float32)
  t0_ref[...] = jnp.broadcast_to(t0, (b, 16))


def _run_k1(query_input, W_enc, b_enc, W0, b0, memory_keys, b):
  return pl.pallas_call(
      _k1_body,
      grid=(NKT,),
      in_specs=[
          pl.BlockSpec((b, D), lambda i: (0, 0)),
          pl.BlockSpec((D, D), lambda i: (0, 0)),
          pl.BlockSpec((1, D), lambda i: (0, 0)),
          pl.BlockSpec((U, D), lambda i: (0, 0)),
          pl.BlockSpec((1, U), lambda i: (0, 0)),
          pl.BlockSpec((KT, D), lambda i: (i, 0)),
      ],
      out_specs=[
          pl.BlockSpec((b, KT), lambda i: (0, i)),
          pl.BlockSpec((1, b, KT // CW), lambda i: (i, 0, 0)),
          pl.BlockSpec((b, D), lambda i: (0, 0)),
          pl.BlockSpec((b, U), lambda i: (0, 0)),
      ],
      out_shape=[
          jax.ShapeDtypeStruct((b, KP), jnp.float32),
          jax.ShapeDtypeStruct((NKT, b, KT // CW), jnp.float32),
          jax.ShapeDtypeStruct((b, D), jnp.float32),
          jax.ShapeDtypeStruct((b, U), jnp.float32),
      ],
      scratch_shapes=[
          pltpu.VMEM((b, D), jnp.float32),
      ],
      compiler_params=pltpu.CompilerParams(
          dimension_semantics=("arbitrary",)),
  )(query_input, W_enc, b_enc, W0, b0, memory_keys)


def _run_k1b(cm, b):
  return pl.pallas_call(
      _k1b_body,
      in_specs=[pl.BlockSpec((b, C), lambda: (0, 0))],
      out_specs=pl.BlockSpec((b, 16), lambda: (0, 0)),
      out_shape=jax.ShapeDtypeStruct((b, 16), jnp.float32),
  )(cm)


_INT_MIN = -2147483648


def _sc_body(sim2, cm, t0, qt, keys, att_out,
             cm_v, t0_v, qt_v, selid_v, chunkbuf, candk_v, candp_v,
             keysbuf, idx_v, att_v, sem):
  rpw = att_v.shape[0]
  cix = lax.axis_index("c")
  six = lax.axis_index("s")
  wid = six * SC_CORES + cix
  base = wid * rpw

  pltpu.sync_copy(cm.at[pl.ds(base, rpw)], cm_v)
  pltpu.sync_copy(t0.at[pl.ds(base, rpw)], t0_v)
  pltpu.sync_copy(qt.at[pl.ds(base, rpw)], qt_v)

  iota16 = lax.iota(jnp.int32, 16)
  zero16i = jnp.zeros((16,), jnp.int32)

  def scatter_append(ref, base, x, m):
    pos = base + plsc.cumsum(m.astype(jnp.int32)) - 1
    pos = jnp.maximum(pos, 0)
    plsc.store_scatter(ref, [pos], x, mask=m)

  def row_body(i, _):
    r = base + i
    t0vec = t0_v[i, pl.ds(0, 16)]

    def sel_body(j, nsel):
      v = cm_v[i, pl.ds(j * 16, 16)]
      m = v >= t0vec
      cnt = jnp.sum(m.astype(jnp.int32))

      @pl.when(cnt > 0)
      def _():
        ids = (r * C + j * 16) + iota16
        scatter_append(selid_v, nsel, ids, m)

      return nsel + cnt

    nsel = lax.fori_loop(0, C // 16, sel_body, jnp.int32(0), unroll=False)

    safeid = jnp.full((16,), r * C + (C - 1), jnp.int32)
    for pj in range(4):
      plsc.store_scatter(selid_v, [nsel + pj * 16 + iota16], safeid)

    def refine(ncand, thr_f):
      del thr_f
      plsc.store_scatter(candk_v, [ncand + iota16],
                         zero16i + jnp.int32(_INT_MIN))
      nv = (ncand + 15) // 16

      def mm_body(j, c):
        mn, mx = c
        kv = candk_v[pl.ds(j * 16, 16)]
        kvm = jnp.where(kv == jnp.int32(_INT_MIN),
                        jnp.int32(2147483647), kv)
        return jnp.minimum(mn, kvm), jnp.maximum(mx, kv)

      mnv, mxv = lax.fori_loop(
          0, nv, mm_body,
          (jnp.full((16,), 2147483647, jnp.int32),
           jnp.full((16,), _INT_MIN, jnp.int32)), unroll=False)
      kmin = jnp.min(mnv)
      kmax = jnp.max(mxv)

      def bit_step(bi, t):
        step = jnp.int32(1) << (30 - bi)

        def probe(t):
          candt = t + step

          def cnt_body(j, cacc):
            kv = candk_v[pl.ds(j * 16, 16)]
            return cacc + (kv >= candt).astype(jnp.int32)

          cacc = lax.fori_loop(0, nv, cnt_body, zero16i, unroll=False)
          cnt = jnp.sum(cacc)
          return jnp.where(cnt >= TOPK, candt, t)

        return lax.cond(step <= kmax - t, probe, lambda t: t, t)

      t = lax.fori_loop(0, 31, bit_step, kmin, unroll=False)

      def cgt_body(j, c):
        kv = candk_v[pl.ds(j * 16, 16)]
        return c + (kv >= t + 1).astype(jnp.int32)

      c_gt = jnp.sum(lax.fori_loop(0, nv, cgt_body, zero16i, unroll=False))
      fill = TOPK - c_gt

      def extract_body(j, carry):
        woff, eqcnt = carry
        kv = candk_v[pl.ds(j * 16, 16)]
        pv = candp_v[pl.ds(j * 16, 16)]
        gt = kv >= (t + 1)
        eq = kv == t
        pos = plsc.cumsum(eq.astype(jnp.int32)) + eqcnt
        keep = gt | (eq & (pos <= fill))
        cnt = jnp.sum(keep.astype(jnp.int32))

        @pl.when(cnt > 0)
        def _():
          scatter_append(candk_v, woff, kv, keep)
          scatter_append(candp_v, woff, pv, keep)

        return woff + cnt, eqcnt + jnp.sum(eq.astype(jnp.int32))

      _, _ = lax.fori_loop(0, nv, extract_body,
                           (jnp.int32(0), jnp.int32(0)), unroll=False)
      newthr = lax.bitcast_convert_type(
          _f32_to_key(jnp.zeros((16,), jnp.int32) + t), jnp.float32)
      return jnp.int32(TOPK), newthr

    def batch_cond(carry):
      off, ncand, thr_f = carry
      return off < nsel

    def batch_body(carry):
      off, ncand, thr_f = carry
      aoff = pl.multiple_of(off, GB)
      pltpu.async_copy(sim2.at[selid_v.at[pl.ds(aoff, GB)]], chunkbuf,
                       sem).wait()

      def chunk_body(g, inner):
        carry2 = inner
        for kk in range(VPC):
          v = chunkbuf[g, pl.ds(kk * 16, 16)]
          hit = jnp.any(v >= carry2[1])

          def do_store(args, v=v, kk=kk):
            ncand, thr_f = args
            ncand, thr_f = lax.cond(ncand + 16 > CAND, refine,
                                    lambda n, tf: (n, tf), ncand, thr_f)
            m2 = v >= thr_f
            kv = _f32_to_key(lax.bitcast_convert_type(v, jnp.int32))
            pv = (off + g) * CW + kk * 16 + iota16
            scatter_append(candk_v, ncand, kv, m2)
            scatter_append(candp_v, ncand, pv, m2)
            return ncand + jnp.sum(m2.astype(jnp.int32)), thr_f

          carry2 = lax.cond(hit, do_store, lambda a: a, carry2)
        return carry2

      ncand, thr_f = lax.fori_loop(0, GB, chunk_body, (ncand, thr_f),
                                   unroll=False)
      return off + GB, ncand, thr_f

    thr0 = t0vec
    off0 = jnp.int32(0)
    _, ncand, thr_f = lax.while_loop(
        batch_cond, batch_body, (off0, jnp.int32(0), thr0))

    ncand, thr_f = refine(ncand, thr_f)

    for dd in range(8):
      p = candp_v[pl.ds(dd * 16, 16)]
      slot = p >> CSH
      lane = p & jnp.int32(CW - 1)
      cid = plsc.load_gather(selid_v, [slot]) - r * C
      idx_v[pl.ds(dd * 16, 16)] = cid * CW + lane

    pltpu.async_copy(keys.at[idx_v], keysbuf, sem).wait()

    qtv = [qt_v[i, pl.ds(dd * 16, 16)] for dd in range(8)]

    svs = []
    for kg in range(8):
      def score_body(j, sv, kg=kg):
        kk2 = kg * 16 + j
        acc = qtv[0] * keysbuf[kk2, pl.ds(0, 16)]
        for dd in range(1, 8):
          acc = acc + qtv[dd] * keysbuf[kk2, pl.ds(dd * 16, 16)]
        s = jnp.sum(acc)
        return jnp.where(iota16 == j, s, sv)

      svs.append(lax.fori_loop(0, 16, score_body,
                               jnp.zeros((16,), jnp.float32), unroll=4))

    mxv = svs[0]
    for kg in range(1, 8):
      mxv = jnp.maximum(mxv, svs[kg])
    mxv = jnp.zeros((16,), jnp.float32) + jnp.max(mxv)

    evs = [jnp.exp(sv - mxv) for sv in svs]
    ssum = evs[0]
    for kg in range(1, 8):
      ssum = ssum + evs[kg]
    rzv = 1.0 / (jnp.zeros((16,), jnp.float32) + jnp.sum(ssum))
    wvs = [ev * rzv for ev in evs]

    accs = tuple(jnp.zeros((16,), jnp.float32) for _ in range(8))
    for kg in range(8):
      def att_body(j, accs, kg=kg):
        kk2 = kg * 16 + j
        ws = jnp.take(wvs[kg], jnp.full((16,), j, jnp.int32))
        return tuple(accs[dd] + ws * keysbuf[kk2, pl.ds(dd * 16, 16)]
                     for dd in range(8))

      accs = lax.fori_loop(0, 16, att_body, accs, unroll=4)
    for dd in range(8):
      att_v[i, pl.ds(dd * 16, 16)] = accs[dd]
    return 0

  lax.fori_loop(0, rpw, row_body, 0, unroll=False)
  pltpu.sync_copy(att_v, att_out.at[pl.ds(base, rpw)])


def _run_sc(sim2, cm, t0, qt, memory_keys, b):
  rpw = b // NW
  mesh = plsc.VectorSubcoreMesh(
      core_axis_name="c", subcore_axis_name="s",
      num_cores=SC_CORES, num_subcores=SC_SUBCORES)
  fn = pl.kernel(
      _sc_body,
      out_type=jax.ShapeDtypeStruct((b, D), jnp.float32),
      mesh=mesh,
      compiler_params=pltpu.CompilerParams(needs_layout_passes=False),
      scratch_types=[
          pltpu.VMEM((rpw, C), jnp.float32),
          pltpu.VMEM((rpw, 16), jnp.float32),
          pltpu.VMEM((rpw, U), jnp.float32),
          pltpu.VMEM((C + GB,), jnp.int32),
          pltpu.VMEM((GB, CW), jnp.float32),
          pltpu.VMEM((CAND + 16,), jnp.int32),
          pltpu.VMEM((CAND + 16,), jnp.int32),
          pltpu.VMEM((TOPK, D), jnp.float32),
          pltpu.VMEM((TOPK,), jnp.int32),
          pltpu.VMEM((rpw, D), jnp.float32),
          pltpu.SemaphoreType.DMA,
      ],
  )
  return fn(sim2, cm, t0, qt, memory_keys)


def _k3_body(qemb_ref, att_ref, g_ref, b_ref, wcls_ref, bcls_ref, out_ref):
  qe = qemb_ref[...]
  x = att_ref[...] + qe
  mu = jnp.mean(x, axis=1, keepdims=True)
  xc = x - mu
  var = jnp.mean(xc * xc, axis=1, keepdims=True)
  ln = xc * lax.rsqrt(var + 1e-5) * g_ref[...] + b_ref[...]
  merged = jnp.concatenate([qe, ln], axis=1)
  out_ref[...] = lax.dot_general(
      merged, wcls_ref[...], (((1,), (1,)), ((), ()))) + bcls_ref[...]


def _run_k3(qemb, att, ln_gamma, ln_beta, W_cls, b_cls, b):
  return pl.pallas_call(
      _k3_body,
      in_specs=[
          pl.BlockSpec((b, D), lambda: (0, 0)),
          pl.BlockSpec((b, D), lambda: (0, 0)),
          pl.BlockSpec((1, D), lambda: (0, 0)),
          pl.BlockSpec((1, D), lambda: (0, 0)),
          pl.BlockSpec((NC, 2 * D), lambda: (0, 0)),
          pl.BlockSpec((1, NC), lambda: (0, 0)),
      ],
      out_specs=pl.BlockSpec((b, NC), lambda: (0, 0)),
      out_shape=jax.ShapeDtypeStruct((b, NC), jnp.float32),
  )(qemb, att, ln_gamma.reshape(1, D), ln_beta.reshape(1, D),
    W_cls, b_cls.reshape(1, NC))


NSPLIT = 4


def kernel(visual_input, query_input, memory_keys, W_enc, b_enc, W0, b0,
           ln_gamma, ln_beta, W_cls, b_cls):
  del visual_input
  bh = B // NSPLIT
  outs = []
  for h in range(NSPLIT):
    q = lax.slice_in_dim(query_input, h * bh, (h + 1) * bh, axis=0)
    sim, cm3, qemb, qt = _run_k1(
        q, W_enc, b_enc.reshape(1, D), W0, b0.reshape(1, U),
        memory_keys, bh)
    cm = cm3.transpose(1, 0, 2).reshape(bh, C)
    t0 = _run_k1b(cm, bh)
    sim2 = sim.reshape(bh * C, CW)
    att = _run_sc(sim2, cm, t0, qt, memory_keys, bh)
    outs.append(_run_k3(qemb, att, ln_gamma, ln_beta, W_cls, b_cls, bh))
  return jnp.concatenate(outs, axis=0)
